# baseline (speedup 1.0000x reference)
import jax
import jax.numpy as jnp
from jax import lax
from jax.experimental import pallas as pl
from jax.experimental.pallas import tpu as pltpu
from jax.experimental.pallas import tpu_sc as plsc

_VOCAB = 1000000
_EMBED = 64
_OUT = 128
_BATCH = 4096
_SEQ = 200

_NC = 2
_NS = 16
_NW = _NC * _NS
_BPW = _BATCH // _NW
_IDXW = _BPW * _SEQ
_CH0 = 104
_CH1 = _SEQ - _CH0


def _bcast_lane(v, j):
    return lax.gather(
        v, jnp.full((16, 1), j, jnp.int32),
        dimension_numbers=lax.GatherDimensionNumbers(
            offset_dims=(), collapsed_slice_dims=(0,), start_index_map=(0,)),
        slice_sizes=(1,),
        mode=lax.GatherScatterMode.PROMISE_IN_BOUNDS)


def _pool_body(x_hbm, table_hbm, out_hbm, idx_v, kidx0, kidx1, rows0, rows1,
               out_v, sem0, sem1):
    wid = lax.axis_index("s") * _NC + lax.axis_index("c")
    idx_base = wid * _IDXW

    pltpu.sync_copy(x_hbm.at[pl.ds(idx_base, _IDXW)],
                    idx_v.at[pl.ds(0, _IDXW)])

    def fire(e, kidx, rows_ref, sem):
        off = e * _SEQ
        for m in range(12):
            kidx[pl.ds(16 * m, 16)] = lax.shift_right_logical(
                idx_v[pl.ds(off + 16 * m, 16)], 1)
        kidx[pl.ds(_SEQ - 16, 16)] = lax.shift_right_logical(
            idx_v[pl.ds(off + _SEQ - 16, 16)], 1)
        pltpu.async_copy(
            table_hbm.at[kidx.at[pl.ds(0, _CH0)]],
            rows_ref.at[pl.ds(0, _CH0)], sem)
        pltpu.async_copy(
            table_hbm.at[kidx.at[pl.ds(_CH0, _CH1)]],
            rows_ref.at[pl.ds(_CH0, _CH1)], sem)

    def wait(rows_ref, sem):
        pltpu.make_async_copy(
            table_hbm.at[pl.ds(0, _SEQ)], rows_ref, sem).wait()

    def accum(rows_ref, e):
        off = e * _SEQ

        def rows16(g, carry, nrows):
            pv = idx_v[pl.ds(off + 16 * g, 16)] & 1
            for j in range(nrows):
                s = 16 * g + j
                pf = _bcast_lane(pv, j).astype(jnp.float32)
                nxt = []
                for m in range(4):
                    left = rows_ref[s, pl.ds(16 * m, 16)]
                    right = rows_ref[s, pl.ds(64 + 16 * m, 16)]
                    nxt.append(carry[m] + left + pf * (right - left))
                carry = tuple(nxt)
            return carry

        z = jnp.zeros((16,), jnp.float32)
        a0, a1, a2, a3 = lax.fori_loop(
            0, _SEQ // 16, lambda g, c: rows16(g, c, 16), (z, z, z, z))
        a0, a1, a2, a3 = rows16(_SEQ // 16, (a0, a1, a2, a3), _SEQ % 16)
        out_v[e, pl.ds(0, 16)] = a0
        out_v[e, pl.ds(16, 16)] = a1
        out_v[e, pl.ds(32, 16)] = a2
        out_v[e, pl.ds(48, 16)] = a3

    fire(0, kidx0, rows0, sem0)
    fire(1, kidx1, rows1, sem1)

    def step(k, _):
        e0 = 2 * k
        wait(rows0, sem0)
        accum(rows0, e0)

        @pl.when(k < _BPW // 2 - 1)
        def _():
            fire(e0 + 2, kidx0, rows0, sem0)

        wait(rows1, sem1)
        accum(rows1, e0 + 1)

        @pl.when(k < _BPW // 2 - 1)
        def _():
            fire(e0 + 3, kidx1, rows1, sem1)

        return 0

    lax.fori_loop(0, _BPW // 2, step, 0)

    pltpu.sync_copy(out_v, out_hbm.at[pl.ds(wid * _BPW, _BPW)])


_pool = pl.kernel(
    _pool_body,
    out_type=jax.ShapeDtypeStruct((_BATCH, _EMBED), jnp.float32),
    mesh=plsc.VectorSubcoreMesh(core_axis_name="c", subcore_axis_name="s",
                                num_cores=_NC, num_subcores=_NS),
    compiler_params=pltpu.CompilerParams(use_tc_tiling_on_sc=True),
    scratch_types=[
        pltpu.VMEM((_IDXW + 16,), jnp.int32),
        pltpu.VMEM((_SEQ,), jnp.int32),
        pltpu.VMEM((_SEQ,), jnp.int32),
        pltpu.VMEM((_SEQ, 2 * _EMBED), jnp.float32),
        pltpu.VMEM((_SEQ, 2 * _EMBED), jnp.float32),
        pltpu.VMEM((_BPW, _EMBED), jnp.float32),
        pltpu.SemaphoreType.DMA,
        pltpu.SemaphoreType.DMA,
    ],
)


def _mlp_body(p_ref, w_ref, b_ref, o_ref):
    pooled = p_ref[...] * (1.0 / _SEQ)
    o_ref[...] = (
        jnp.dot(pooled, w_ref[...], preferred_element_type=jnp.float32)
        + b_ref[...])


_mlp = pl.pallas_call(
    _mlp_body,
    out_shape=jax.ShapeDtypeStruct((_BATCH, _OUT), jnp.float32),
)


@jax.jit
def kernel(x, table, W1, b1):
    x_flat = x.reshape(-1).astype(jnp.int32)
    table_pairs = table.reshape(_VOCAB // 2, 2 * _EMBED)
    sums = _pool(x_flat, table_pairs)
    return _mlp(sums, W1, b1.reshape(1, _OUT))
